# Initial kernel scaffold; baseline (speedup 1.0000x reference)
import functools

import jax
import jax.numpy as jnp
from jax import lax
from jax.experimental import pallas as pl
from jax.experimental.pallas import tpu as pltpu
from jax.experimental.pallas import tpu_sc as plsc

_N = 10000
_E = 320000
_IN = 128
_HID = 64

_NTILES = 16
_EPT = _E // _NTILES
_SL = 125
_K = 8
_CH = _SL * _K
_NCH = _EPT // _CH
_RPT = _N // _NTILES
_DW = 16


_BM_A = 2000


def _proj_body(x_ref, wp_ref, we_ref, bp_ref, out_ref):
    xb = x_ref[0]
    we = we_ref[0]
    m = jnp.dot(wp_ref[...], we, preferred_element_type=jnp.float32)
    c = jnp.dot(bp_ref[...], we, preferred_element_type=jnp.float32)
    out_ref[...] = jnp.dot(xb, m, preferred_element_type=jnp.float32) + c


def _build_table(x_stacked, w_proj, w_enc_stacked, b_proj_2d):
    nb = _N // _BM_A
    return pl.pallas_call(
        _proj_body,
        grid=(2, 2, nb),
        in_specs=[
            pl.BlockSpec((1, _BM_A, _IN), lambda v, r, i: (v, i, 0)),
            pl.BlockSpec((_IN, _IN), lambda v, r, i: (0, 0)),
            pl.BlockSpec((1, _IN, _HID), lambda v, r, i: (r, 0, 0)),
            pl.BlockSpec((1, _IN), lambda v, r, i: (0, 0)),
        ],
        out_specs=pl.BlockSpec(
            (_BM_A, _HID), lambda v, r, i: ((v * 2 + r) * nb + i, 0)),
        out_shape=jax.ShapeDtypeStruct((4 * _N, _HID), jnp.float32),
    )(x_stacked, w_proj, w_enc_stacked, b_proj_2d)


def _sc_body(table, src_h, dst_h, zrows, zdeg, ones_h,
             sa, sb, da, db,
             acc0, acc1, deg0, deg1, idx_s, idx_d, rows, ones, zd, sem):
    c = lax.axis_index("c")
    s = lax.axis_index("s")
    r0 = s * _RPT

    pltpu.sync_copy(zrows, rows.at[pl.ds(0, _RPT), :])
    pltpu.sync_copy(zdeg, zd)
    pltpu.sync_copy(ones_h, ones)
    pltpu.sync_copy(rows.at[pl.ds(0, _RPT), :], acc0.at[pl.ds(r0, _RPT), :])
    pltpu.sync_copy(rows.at[pl.ds(0, _RPT), :], acc1.at[pl.ds(r0, _RPT), :])
    pltpu.sync_copy(zd, deg0.at[pl.ds(r0, _RPT), :])
    pltpu.sync_copy(zd, deg1.at[pl.ds(r0, _RPT), :])
    plsc.subcore_barrier()

    for r, (acc, deg) in enumerate(((acc0, deg0), (acc1, deg1))):
        def chunk_body(k, unused, r=r, acc=acc, deg=deg):
            q = s * _NCH + k
            pltpu.sync_copy(src_h.at[c, r, q], idx_s)
            pltpu.sync_copy(dst_h.at[c, r, q], idx_d)
            cps = [
                pltpu.async_copy(table.at[idx_s.at[j]],
                                 rows.at[pl.ds(j * _SL, _SL), :], sem)
                for j in range(_K)
            ]
            for cp in cps:
                cp.wait()
            for j in range(_K):
                pltpu.sync_copy(rows.at[pl.ds(j * _SL, _SL), :],
                                acc.at[idx_d.at[j]], add=True)
                pltpu.sync_copy(ones, deg.at[idx_d.at[j]], add=True)
            return unused

        lax.fori_loop(0, _NCH, chunk_body, 0)

    plsc.subcore_barrier()

    for r, (acc, deg) in enumerate(((acc0, deg0), (acc1, deg1))):
        pltpu.sync_copy(acc.at[pl.ds(r0, _RPT), :], rows.at[pl.ds(0, _RPT), :])
        pltpu.sync_copy(deg.at[pl.ds(r0, _RPT), :], zd)

        @pl.when(c == 0)
        def _():
            pltpu.sync_copy(rows.at[pl.ds(0, _RPT), :],
                            sa.at[r, pl.ds(r0, _RPT), :])
            pltpu.sync_copy(zd, da.at[r, pl.ds(r0, _RPT), :])

        @pl.when(c == 1)
        def _():
            pltpu.sync_copy(rows.at[pl.ds(0, _RPT), :],
                            sb.at[r, pl.ds(r0, _RPT), :])
            pltpu.sync_copy(zd, db.at[r, pl.ds(r0, _RPT), :])


def _segment_sums(table, src5, dst5, zrows, zdeg, ones_h):
    f32 = jnp.float32
    run = pl.kernel(
        _sc_body,
        out_type=[
            jax.ShapeDtypeStruct((2, _N, _HID), f32),
            jax.ShapeDtypeStruct((2, _N, _HID), f32),
            jax.ShapeDtypeStruct((2, _N, _DW), f32),
            jax.ShapeDtypeStruct((2, _N, _DW), f32),
        ],
        mesh=plsc.VectorSubcoreMesh(core_axis_name="c", subcore_axis_name="s"),
        scratch_types=[
            pltpu.VMEM_SHARED((_N, _HID), f32),
            pltpu.VMEM_SHARED((_N, _HID), f32),
            pltpu.VMEM_SHARED((_N, _DW), f32),
            pltpu.VMEM_SHARED((_N, _DW), f32),
            pltpu.VMEM((_K, _SL), jnp.int32),
            pltpu.VMEM((_K, _SL), jnp.int32),
            pltpu.VMEM((_CH, _HID), f32),
            pltpu.VMEM((_SL, _DW), f32),
            pltpu.VMEM((_RPT, _DW), f32),
            pltpu.SemaphoreType.DMA,
        ],
    )
    return run(table, src5, dst5, zrows, zdeg, ones_h)


_BM_C = 2000


def _fin_body(s_ref, d_ref, benc_ref, attn_ref, wdec_ref, bdec_ref,
              z_ref, xh_ref):
    a0 = attn_ref[0]
    a1 = attn_ref[1]
    m = jnp.maximum(a0, a1)
    e0 = jnp.exp(a0 - m)
    e1 = jnp.exp(a1 - m)
    w0 = e0 / (e0 + e1)
    w1 = e1 / (e0 + e1)
    d0 = jnp.maximum(d_ref[0, :, 0:1], 1.0)
    d1 = jnp.maximum(d_ref[1, :, 0:1], 1.0)
    z0 = jnp.maximum(s_ref[0] / d0 + benc_ref[0:1, :], 0.0)
    z1 = jnp.maximum(s_ref[1] / d1 + benc_ref[1:2, :], 0.0)
    z = w0 * z0 + w1 * z1
    z_ref[...] = z
    xh_ref[...] = (jnp.dot(z, wdec_ref[...], preferred_element_type=jnp.float32)
                   + bdec_ref[0:1, :])


def _finalize(s_v, d_v, benc, attn, w_dec, bdec_2d):
    nb = _N // _BM_C
    return pl.pallas_call(
        _fin_body,
        grid=(nb,),
        in_specs=[
            pl.BlockSpec((2, _BM_C, _HID), lambda i: (0, i, 0)),
            pl.BlockSpec((2, _BM_C, _DW), lambda i: (0, i, 0)),
            pl.BlockSpec((2, _HID), lambda i: (0, 0)),
            pl.BlockSpec(memory_space=pltpu.SMEM),
            pl.BlockSpec((_HID, _IN), lambda i: (0, 0)),
            pl.BlockSpec((1, _IN), lambda i: (0, 0)),
        ],
        out_specs=[
            pl.BlockSpec((_BM_C, _HID), lambda i: (i, 0)),
            pl.BlockSpec((_BM_C, _IN), lambda i: (i, 0)),
        ],
        out_shape=[
            jax.ShapeDtypeStruct((_N, _HID), jnp.float32),
            jax.ShapeDtypeStruct((_N, _IN), jnp.float32),
        ],
    )(s_v, d_v, benc, attn, w_dec, bdec_2d)


_BM_D = 512
_BN_D = 2048


def _adj_body(zi_ref, zj_ref, out_ref):
    x = lax.dot_general(zi_ref[...], zj_ref[...],
                        dimension_numbers=(((1,), (1,)), ((), ())),
                        preferred_element_type=jnp.float32)
    out_ref[...] = 1.0 / (1.0 + jnp.exp(-x))


def _adjacency(z):
    ni = pl.cdiv(_N, _BM_D)
    nj = pl.cdiv(_N, _BN_D)
    return pl.pallas_call(
        _adj_body,
        grid=(ni, nj),
        in_specs=[
            pl.BlockSpec((_BM_D, _HID), lambda i, j: (i, 0)),
            pl.BlockSpec((_BN_D, _HID), lambda i, j: (j, 0)),
        ],
        out_specs=pl.BlockSpec((_BM_D, _BN_D), lambda i, j: (i, j)),
        out_shape=jax.ShapeDtypeStruct((_N, _N), jnp.float32),
    )(z, z)


def kernel(x_view_A, edge_indices_A, x_view_B, edge_indices_B, W_proj, b_proj,
           W_enc0, b_enc0, W_enc1, b_enc1, attn_weights, W_dec, b_dec):
    f32 = jnp.float32

    x_stacked = jnp.stack([x_view_A, x_view_B])
    w_enc_stacked = jnp.stack([W_enc0, W_enc1])
    b_proj_2d = b_proj.reshape(1, _IN)
    bdec_2d = b_dec.reshape(1, _IN)
    benc = jnp.stack([b_enc0, b_enc1])

    ei = jnp.stack([edge_indices_A, edge_indices_B])
    table_off = (jnp.arange(2, dtype=jnp.int32)[:, None, None] * 2
                 + jnp.arange(2, dtype=jnp.int32)[None, :, None]) * _N
    src = ei[:, :, 0, :] + table_off
    dst = ei[:, :, 1, :]
    nchunks = _NTILES * _NCH
    src5 = src.reshape(2, 2, nchunks, _K, _SL)
    dst5 = dst.reshape(2, 2, nchunks, _K, _SL)

    zrows = jnp.zeros((_RPT, _HID), f32)
    zdeg = jnp.zeros((_RPT, _DW), f32)
    ones_h = jnp.zeros((_SL, _DW), f32).at[:, 0].set(1.0)

    table = _build_table(x_stacked, W_proj, w_enc_stacked, b_proj_2d)

    sa, sb, da, db = _segment_sums(table, src5, dst5, zrows, zdeg, ones_h)

    z_A, xh_A = _finalize(sa, da, benc, attn_weights, W_dec, bdec_2d)
    z_B, xh_B = _finalize(sb, db, benc, attn_weights, W_dec, bdec_2d)

    adj_A = _adjacency(z_A)
    adj_B = _adjacency(z_B)

    return ((xh_A, adj_A), (xh_B, adj_B), (z_A, z_B))


# speedup vs baseline: 4.7394x; 4.7394x over previous
import jax
import jax.numpy as jnp
from jax import lax
from jax.experimental import pallas as pl
from jax.experimental.pallas import tpu as pltpu
from jax.experimental.pallas import tpu_sc as plsc

_N = 10000
_E = 320000
_IN = 128
_HID = 64

_NTILES = 16
_SL = 128
_K = 8
_CH = _SL * _K
_NCH = 20
_EPT = _CH * _NCH
_E2 = _EPT * _NTILES
_NPAD = 10240
_RPT = _NPAD // _NTILES


_BM_A = 2000


def _proj_body(x_ref, wp_ref, we_ref, bp_ref, out_ref):
    xb = x_ref[0]
    f32 = jnp.float32
    m0 = jnp.dot(wp_ref[...], we_ref[0], preferred_element_type=f32)
    m1 = jnp.dot(wp_ref[...], we_ref[1], preferred_element_type=f32)
    c0 = jnp.dot(bp_ref[...], we_ref[0], preferred_element_type=f32)
    c1 = jnp.dot(bp_ref[...], we_ref[1], preferred_element_type=f32)
    g0 = jnp.dot(xb, m0, preferred_element_type=f32) + c0
    g1 = jnp.dot(xb, m1, preferred_element_type=f32) + c1
    out_ref[...] = jnp.concatenate([g0, g1], axis=1)


def _build_table(x_stacked, w_proj, w_enc_stacked, b_proj_2d):
    nb = _N // _BM_A
    return pl.pallas_call(
        _proj_body,
        grid=(2, nb),
        in_specs=[
            pl.BlockSpec((1, _BM_A, _IN), lambda v, i: (v, i, 0)),
            pl.BlockSpec((_IN, _IN), lambda v, i: (0, 0)),
            pl.BlockSpec((2, _IN, _HID), lambda v, i: (0, 0, 0)),
            pl.BlockSpec((1, _IN), lambda v, i: (0, 0)),
        ],
        out_specs=pl.BlockSpec((_BM_A, 2 * _HID), lambda v, i: (v * nb + i, 0)),
        out_shape=jax.ShapeDtypeStruct((2 * _N, 2 * _HID), jnp.float32),
    )(x_stacked, w_proj, w_enc_stacked, b_proj_2d)


def _sc_body(table, src_h, dst_h, zrows, zdeg,
             s_out, d_out,
             acc, deg_all, idx_s, idx_d, buf0, buf1, ldeg, dbuf, dres, sem):
    c = lax.axis_index("c")
    s = lax.axis_index("s")
    r0 = s * _RPT
    bufs = (buf0, buf1)
    fones = jnp.full((16,), 1.0, jnp.float32)

    for r in range(2):
        pltpu.sync_copy(zrows, acc.at[pl.ds(r0, _RPT), :])
        pltpu.sync_copy(zdeg, ldeg)
        plsc.subcore_barrier()

        def chunk_body(k, unused, r=r):
            q = s * _NCH + k
            pltpu.sync_copy(src_h.at[c, r, q], idx_s)
            pltpu.sync_copy(dst_h.at[c, r, q], idx_d)
            cps = {0: pltpu.async_copy(table.at[idx_s.at[0]], bufs[0], sem)}
            for j in range(_K):
                if j + 1 < _K:
                    cps[j + 1] = pltpu.async_copy(
                        table.at[idx_s.at[j + 1]], bufs[(j + 1) % 2], sem)
                for t in range(_SL // 16):
                    iv = idx_d[j, pl.ds(t * 16, 16)]
                    plsc.addupdate_scatter(ldeg, [iv], fones)
                cps[j].wait()
                pltpu.sync_copy(bufs[j % 2], acc.at[idx_d.at[j]], add=True)
            return unused

        lax.fori_loop(0, _NCH, chunk_body, 0)

        pltpu.sync_copy(ldeg, deg_all.at[s])
        plsc.subcore_barrier()
        pltpu.sync_copy(deg_all.at[:, pl.ds(r0, _RPT)], dbuf)
        for g in range(_RPT // 16):
            acc16 = dbuf[0, pl.ds(g * 16, 16)]
            for t in range(1, _NTILES):
                acc16 = acc16 + dbuf[t, pl.ds(g * 16, 16)]
            dres[pl.ds(g * 16, 16)] = acc16

        pltpu.sync_copy(acc.at[pl.ds(r0, _RPT), :],
                        s_out.at[c, r, pl.ds(r0, _RPT), :])
        pltpu.sync_copy(dres, d_out.at[c, r, pl.ds(r0, _RPT)])
        plsc.subcore_barrier()


def _segment_sums(table64, src5, dst5, zrows, zdeg):
    f32 = jnp.float32
    run = pl.kernel(
        _sc_body,
        out_type=[
            jax.ShapeDtypeStruct((2, 2, _NPAD, _HID), f32),
            jax.ShapeDtypeStruct((2, 2, _NPAD), f32),
        ],
        mesh=plsc.VectorSubcoreMesh(core_axis_name="c", subcore_axis_name="s"),
        compiler_params=pltpu.CompilerParams(use_tc_tiling_on_sc=False,
                                             needs_layout_passes=False),
        scratch_types=[
            pltpu.VMEM_SHARED((_NPAD, _HID), f32),
            pltpu.VMEM_SHARED((_NTILES, _NPAD), f32),
            pltpu.VMEM((_K, _SL), jnp.int32),
            pltpu.VMEM((_K, _SL), jnp.int32),
            pltpu.VMEM((_SL, _HID), f32),
            pltpu.VMEM((_SL, _HID), f32),
            pltpu.VMEM((_NPAD,), f32),
            pltpu.VMEM((_NTILES, _RPT), f32),
            pltpu.VMEM((_RPT,), f32),
            pltpu.SemaphoreType.DMA,
        ],
    )
    return run(table64, src5, dst5, zrows, zdeg)


_BM_C = 2000


def _fin_body(s_ref, d_ref, benc_ref, attn_ref, wdec_ref, bdec_ref,
              z_ref, xh_ref):
    a0 = attn_ref[0]
    a1 = attn_ref[1]
    m = jnp.maximum(a0, a1)
    e0 = jnp.exp(a0 - m)
    e1 = jnp.exp(a1 - m)
    w0 = e0 / (e0 + e1)
    w1 = e1 / (e0 + e1)
    sr = s_ref[0]
    dr = d_ref[0]
    d0 = jnp.maximum(dr[0], 1.0)
    d1 = jnp.maximum(dr[1], 1.0)
    z0 = jnp.maximum(sr[0] / d0 + benc_ref[0:1, :], 0.0)
    z1 = jnp.maximum(sr[1] / d1 + benc_ref[1:2, :], 0.0)
    z = w0 * z0 + w1 * z1
    z_ref[...] = z
    xh_ref[...] = (jnp.dot(z, wdec_ref[...], preferred_element_type=jnp.float32)
                   + bdec_ref[0:1, :])


def _finalize(s_all, d_all, view, benc, attn, w_dec, bdec_2d):
    nb = _N // _BM_C
    return pl.pallas_call(
        _fin_body,
        grid=(nb,),
        in_specs=[
            pl.BlockSpec((1, 2, _BM_C, _HID), lambda i, v=view: (v, 0, i, 0)),
            pl.BlockSpec((1, 2, _BM_C, 1), lambda i, v=view: (v, 0, i, 0)),
            pl.BlockSpec((2, _HID), lambda i: (0, 0)),
            pl.BlockSpec(memory_space=pltpu.SMEM),
            pl.BlockSpec((_HID, _IN), lambda i: (0, 0)),
            pl.BlockSpec((1, _IN), lambda i: (0, 0)),
        ],
        out_specs=[
            pl.BlockSpec((_BM_C, _HID), lambda i: (i, 0)),
            pl.BlockSpec((_BM_C, _IN), lambda i: (i, 0)),
        ],
        out_shape=[
            jax.ShapeDtypeStruct((_N, _HID), jnp.float32),
            jax.ShapeDtypeStruct((_N, _IN), jnp.float32),
        ],
    )(s_all, d_all, benc, attn, w_dec, bdec_2d)


_BM_D = 512
_BN_D = 2048


def _adj_body(zi_ref, zj_ref, out_ref):
    x = lax.dot_general(zi_ref[...], zj_ref[...],
                        dimension_numbers=(((1,), (1,)), ((), ())),
                        preferred_element_type=jnp.float32)
    out_ref[...] = 1.0 / (1.0 + jnp.exp(-x))


def _adjacency(z):
    ni = pl.cdiv(_N, _BM_D)
    nj = pl.cdiv(_N, _BN_D)
    return pl.pallas_call(
        _adj_body,
        grid=(ni, nj),
        in_specs=[
            pl.BlockSpec((_BM_D, _HID), lambda i, j: (i, 0)),
            pl.BlockSpec((_BN_D, _HID), lambda i, j: (j, 0)),
        ],
        out_specs=pl.BlockSpec((_BM_D, _BN_D), lambda i, j: (i, j)),
        out_shape=jax.ShapeDtypeStruct((_N, _N), jnp.float32),
    )(z, z)


def kernel(x_view_A, edge_indices_A, x_view_B, edge_indices_B, W_proj, b_proj,
           W_enc0, b_enc0, W_enc1, b_enc1, attn_weights, W_dec, b_dec):
    f32 = jnp.float32

    x_stacked = jnp.stack([x_view_A, x_view_B])
    w_enc_stacked = jnp.stack([W_enc0, W_enc1])
    b_proj_2d = b_proj.reshape(1, _IN)
    bdec_2d = b_dec.reshape(1, _IN)
    benc = jnp.stack([b_enc0, b_enc1])

    ei = jnp.stack([edge_indices_A, edge_indices_B])
    voff = jnp.arange(2, dtype=jnp.int32).reshape(2, 1, 1) * (2 * _N)
    roff = jnp.arange(2, dtype=jnp.int32).reshape(1, 2, 1)
    src = 2 * ei[:, :, 0, :] + voff + roff
    dst = ei[:, :, 1, :]
    pad = _E2 - _E
    src = jnp.concatenate(
        [src, jnp.zeros((2, 2, pad), jnp.int32)], axis=2)
    dst = jnp.concatenate(
        [dst, jnp.full((2, 2, pad), _NPAD - 1, jnp.int32)], axis=2)
    nchunks = _NTILES * _NCH
    src5 = src.reshape(2, 2, nchunks, _K, _SL)
    dst5 = dst.reshape(2, 2, nchunks, _K, _SL)

    zrows = jnp.zeros((_RPT, _HID), f32)
    zdeg = jnp.zeros((_NPAD,), f32)

    table = _build_table(x_stacked, W_proj, w_enc_stacked, b_proj_2d)
    table64 = table.reshape(4 * _N, _HID)

    s_all, d_all = _segment_sums(table64, src5, dst5, zrows, zdeg)
    d_all4 = d_all.reshape(2, 2, _NPAD, 1)

    z_A, xh_A = _finalize(s_all, d_all4, 0, benc, attn_weights, W_dec, bdec_2d)
    z_B, xh_B = _finalize(s_all, d_all4, 1, benc, attn_weights, W_dec, bdec_2d)

    adj_A = _adjacency(z_A)
    adj_B = _adjacency(z_B)

    return ((xh_A, adj_A), (xh_B, adj_B), (z_A, z_B))


# speedup vs baseline: 4.7467x; 1.0015x over previous
import jax
import jax.numpy as jnp
from jax import lax
from jax.experimental import pallas as pl
from jax.experimental.pallas import tpu as pltpu
from jax.experimental.pallas import tpu_sc as plsc

_N = 10000
_E = 320000
_IN = 128
_HID = 64

_NTILES = 16
_SL = 128
_K = 8
_CH = _SL * _K
_NCH = 20
_EPT = _CH * _NCH
_E2 = _EPT * _NTILES
_NPAD = 10240
_RPT = _NPAD // _NTILES


_BM_A = 2000


def _proj_body(x_ref, wp_ref, we_ref, bp_ref, out_ref):
    xb = x_ref[0]
    f32 = jnp.float32
    m0 = jnp.dot(wp_ref[...], we_ref[0], preferred_element_type=f32)
    m1 = jnp.dot(wp_ref[...], we_ref[1], preferred_element_type=f32)
    c0 = jnp.dot(bp_ref[...], we_ref[0], preferred_element_type=f32)
    c1 = jnp.dot(bp_ref[...], we_ref[1], preferred_element_type=f32)
    g0 = jnp.dot(xb, m0, preferred_element_type=f32) + c0
    g1 = jnp.dot(xb, m1, preferred_element_type=f32) + c1
    out_ref[...] = jnp.concatenate([g0, g1], axis=1)


def _build_table(x_stacked, w_proj, w_enc_stacked, b_proj_2d):
    nb = _N // _BM_A
    return pl.pallas_call(
        _proj_body,
        grid=(2, nb),
        in_specs=[
            pl.BlockSpec((1, _BM_A, _IN), lambda v, i: (v, i, 0)),
            pl.BlockSpec((_IN, _IN), lambda v, i: (0, 0)),
            pl.BlockSpec((2, _IN, _HID), lambda v, i: (0, 0, 0)),
            pl.BlockSpec((1, _IN), lambda v, i: (0, 0)),
        ],
        out_specs=pl.BlockSpec((_BM_A, 2 * _HID), lambda v, i: (v * nb + i, 0)),
        out_shape=jax.ShapeDtypeStruct((2 * _N, 2 * _HID), jnp.float32),
    )(x_stacked, w_proj, w_enc_stacked, b_proj_2d)


def _sc_body(table, src_h, dst_h, zrows, zdeg, ones_h,
             s_out, d_out,
             acc, deg, idx_s, idx_d, buf0, buf1, ones, sem):
    c = lax.axis_index("c")
    s = lax.axis_index("s")
    r0 = s * _RPT
    bufs = (buf0, buf1)

    pltpu.sync_copy(ones_h, ones)
    for r in range(2):
        pltpu.sync_copy(zrows, acc.at[pl.ds(r0, _RPT), :])
        pltpu.sync_copy(zdeg, deg.at[pl.ds(r0, _RPT)])
        plsc.subcore_barrier()

        def chunk_body(k, unused, r=r):
            q = s * _NCH + k
            pltpu.sync_copy(src_h.at[c, r, q], idx_s)
            pltpu.sync_copy(dst_h.at[c, r, q], idx_d)
            cps = {0: pltpu.async_copy(table.at[idx_s.at[0]], bufs[0], sem)}
            for j in range(_K):
                if j + 1 < _K:
                    cps[j + 1] = pltpu.async_copy(
                        table.at[idx_s.at[j + 1]], bufs[(j + 1) % 2], sem)
                pltpu.sync_copy(ones.at[j], deg.at[idx_d.at[j]], add=True)
                cps[j].wait()
                pltpu.sync_copy(bufs[j % 2], acc.at[idx_d.at[j]], add=True)
            return unused

        lax.fori_loop(0, _NCH, chunk_body, 0)
        plsc.subcore_barrier()

        pltpu.sync_copy(acc.at[pl.ds(r0, _RPT), :],
                        s_out.at[c, r, pl.ds(r0, _RPT), :])
        pltpu.sync_copy(deg.at[pl.ds(r0, _RPT)], d_out.at[c, r, pl.ds(r0, _RPT)])
        plsc.subcore_barrier()


def _segment_sums(table64, src5, dst5, zrows, zdeg, ones_h):
    f32 = jnp.float32
    run = pl.kernel(
        _sc_body,
        out_type=[
            jax.ShapeDtypeStruct((2, 2, _NPAD, _HID), f32),
            jax.ShapeDtypeStruct((2, 2, _NPAD), f32),
        ],
        mesh=plsc.VectorSubcoreMesh(core_axis_name="c", subcore_axis_name="s"),
        compiler_params=pltpu.CompilerParams(use_tc_tiling_on_sc=False,
                                             needs_layout_passes=False),
        scratch_types=[
            pltpu.VMEM_SHARED((_NPAD, _HID), f32),
            pltpu.VMEM_SHARED((_NPAD,), f32),
            pltpu.VMEM((_K, _SL), jnp.int32),
            pltpu.VMEM((_K, _SL), jnp.int32),
            pltpu.VMEM((_SL, _HID), f32),
            pltpu.VMEM((_SL, _HID), f32),
            pltpu.VMEM((_K, _SL), f32),
            pltpu.SemaphoreType.DMA,
        ],
    )
    return run(table64, src5, dst5, zrows, zdeg, ones_h)


_BM_C = 2000


def _fin_body(s_ref, d_ref, benc_ref, attn_ref, wdec_ref, bdec_ref,
              z_ref, xh_ref):
    a0 = attn_ref[0]
    a1 = attn_ref[1]
    m = jnp.maximum(a0, a1)
    e0 = jnp.exp(a0 - m)
    e1 = jnp.exp(a1 - m)
    w0 = e0 / (e0 + e1)
    w1 = e1 / (e0 + e1)
    sr = s_ref[0]
    dr = d_ref[0]
    d0 = jnp.maximum(dr[0], 1.0)
    d1 = jnp.maximum(dr[1], 1.0)
    z0 = jnp.maximum(sr[0] / d0 + benc_ref[0:1, :], 0.0)
    z1 = jnp.maximum(sr[1] / d1 + benc_ref[1:2, :], 0.0)
    z = w0 * z0 + w1 * z1
    z_ref[...] = z
    xh_ref[...] = (jnp.dot(z, wdec_ref[...], preferred_element_type=jnp.float32)
                   + bdec_ref[0:1, :])


def _finalize(s_all, d_all, view, benc, attn, w_dec, bdec_2d):
    nb = _N // _BM_C
    return pl.pallas_call(
        _fin_body,
        grid=(nb,),
        in_specs=[
            pl.BlockSpec((1, 2, _BM_C, _HID), lambda i, v=view: (v, 0, i, 0)),
            pl.BlockSpec((1, 2, _BM_C, 1), lambda i, v=view: (v, 0, i, 0)),
            pl.BlockSpec((2, _HID), lambda i: (0, 0)),
            pl.BlockSpec(memory_space=pltpu.SMEM),
            pl.BlockSpec((_HID, _IN), lambda i: (0, 0)),
            pl.BlockSpec((1, _IN), lambda i: (0, 0)),
        ],
        out_specs=[
            pl.BlockSpec((_BM_C, _HID), lambda i: (i, 0)),
            pl.BlockSpec((_BM_C, _IN), lambda i: (i, 0)),
        ],
        out_shape=[
            jax.ShapeDtypeStruct((_N, _HID), jnp.float32),
            jax.ShapeDtypeStruct((_N, _IN), jnp.float32),
        ],
    )(s_all, d_all, benc, attn, w_dec, bdec_2d)


_BM_D = 512
_BN_D = 2048


def _adj_body(zi_ref, zj_ref, out_ref):
    x = lax.dot_general(zi_ref[...], zj_ref[...],
                        dimension_numbers=(((1,), (1,)), ((), ())),
                        preferred_element_type=jnp.float32)
    out_ref[...] = 1.0 / (1.0 + jnp.exp(-x))


def _adjacency(z):
    ni = pl.cdiv(_N, _BM_D)
    nj = pl.cdiv(_N, _BN_D)
    return pl.pallas_call(
        _adj_body,
        grid=(ni, nj),
        in_specs=[
            pl.BlockSpec((_BM_D, _HID), lambda i, j: (i, 0)),
            pl.BlockSpec((_BN_D, _HID), lambda i, j: (j, 0)),
        ],
        out_specs=pl.BlockSpec((_BM_D, _BN_D), lambda i, j: (i, j)),
        out_shape=jax.ShapeDtypeStruct((_N, _N), jnp.float32),
    )(z, z)


def kernel(x_view_A, edge_indices_A, x_view_B, edge_indices_B, W_proj, b_proj,
           W_enc0, b_enc0, W_enc1, b_enc1, attn_weights, W_dec, b_dec):
    f32 = jnp.float32

    x_stacked = jnp.stack([x_view_A, x_view_B])
    w_enc_stacked = jnp.stack([W_enc0, W_enc1])
    b_proj_2d = b_proj.reshape(1, _IN)
    bdec_2d = b_dec.reshape(1, _IN)
    benc = jnp.stack([b_enc0, b_enc1])

    ei = jnp.stack([edge_indices_A, edge_indices_B])
    voff = jnp.arange(2, dtype=jnp.int32).reshape(2, 1, 1) * (2 * _N)
    roff = jnp.arange(2, dtype=jnp.int32).reshape(1, 2, 1)
    src = 2 * ei[:, :, 0, :] + voff + roff
    dst = ei[:, :, 1, :]
    pad = _E2 - _E
    src = jnp.concatenate(
        [src, jnp.zeros((2, 2, pad), jnp.int32)], axis=2)
    dst = jnp.concatenate(
        [dst, jnp.full((2, 2, pad), _NPAD - 1, jnp.int32)], axis=2)
    nchunks = _NTILES * _NCH
    src5 = src.reshape(2, 2, nchunks, _K, _SL)
    dst5 = dst.reshape(2, 2, nchunks, _K, _SL)

    zrows = jnp.zeros((_RPT, _HID), f32)
    zdeg = jnp.zeros((_RPT,), f32)
    ones_h = jnp.ones((_K, _SL), f32)

    table = _build_table(x_stacked, W_proj, w_enc_stacked, b_proj_2d)
    table64 = table.reshape(4 * _N, _HID)

    s_all, d_all = _segment_sums(table64, src5, dst5, zrows, zdeg, ones_h)
    d_all4 = d_all.reshape(2, 2, _NPAD, 1)

    z_A, xh_A = _finalize(s_all, d_all4, 0, benc, attn_weights, W_dec, bdec_2d)
    z_B, xh_B = _finalize(s_all, d_all4, 1, benc, attn_weights, W_dec, bdec_2d)

    adj_A = _adjacency(z_A)
    adj_B = _adjacency(z_B)

    return ((xh_A, adj_A), (xh_B, adj_B), (z_A, z_B))


# speedup vs baseline: 4.7576x; 1.0023x over previous
import jax
import jax.numpy as jnp
from jax import lax
from jax.experimental import pallas as pl
from jax.experimental.pallas import tpu as pltpu
from jax.experimental.pallas import tpu_sc as plsc

_N = 10000
_E = 320000
_IN = 128
_HID = 64

_NTILES = 16
_CH = 1024
_NCH = 20
_EPT = _CH * _NCH
_E2 = _EPT * _NTILES
_NPAD = 10240
_RPT = _NPAD // _NTILES


_BM_A = 2000


def _proj_body(x_ref, wp_ref, we_ref, bp_ref, out_ref):
    xb = x_ref[0]
    f32 = jnp.float32
    m0 = jnp.dot(wp_ref[...], we_ref[0], preferred_element_type=f32)
    m1 = jnp.dot(wp_ref[...], we_ref[1], preferred_element_type=f32)
    c0 = jnp.dot(bp_ref[...], we_ref[0], preferred_element_type=f32)
    c1 = jnp.dot(bp_ref[...], we_ref[1], preferred_element_type=f32)
    g0 = jnp.dot(xb, m0, preferred_element_type=f32) + c0
    g1 = jnp.dot(xb, m1, preferred_element_type=f32) + c1
    out_ref[...] = jnp.concatenate([g0, g1], axis=1)


def _build_table(x_stacked, w_proj, w_enc_stacked, b_proj_2d):
    nb = _N // _BM_A
    return pl.pallas_call(
        _proj_body,
        grid=(2, nb),
        in_specs=[
            pl.BlockSpec((1, _BM_A, _IN), lambda v, i: (v, i, 0)),
            pl.BlockSpec((_IN, _IN), lambda v, i: (0, 0)),
            pl.BlockSpec((2, _IN, _HID), lambda v, i: (0, 0, 0)),
            pl.BlockSpec((1, _IN), lambda v, i: (0, 0)),
        ],
        out_specs=pl.BlockSpec((_BM_A, 2 * _HID), lambda v, i: (v * nb + i, 0)),
        out_shape=jax.ShapeDtypeStruct((2 * _N, 2 * _HID), jnp.float32),
    )(x_stacked, w_proj, w_enc_stacked, b_proj_2d)


def _sc_body(table, src_h, dst_h, zrows, zdeg, ones_h,
             s_out, d_out,
             acc, deg, idx_s, idx_d, buf, ones, sem):
    c = lax.axis_index("c")
    s = lax.axis_index("s")
    r0 = s * _RPT

    pltpu.sync_copy(ones_h, ones)
    for r in range(2):
        pltpu.sync_copy(zrows, acc.at[pl.ds(r0, _RPT), :])
        pltpu.sync_copy(zdeg, deg.at[pl.ds(r0, _RPT)])
        plsc.subcore_barrier()

        def chunk_body(k, unused, r=r):
            q = s * _NCH + k
            pltpu.sync_copy(src_h.at[c, r, q], idx_s)
            pltpu.sync_copy(dst_h.at[c, r, q], idx_d)
            cp = pltpu.async_copy(table.at[idx_s], buf, sem)
            pltpu.sync_copy(ones, deg.at[idx_d], add=True)
            cp.wait()
            pltpu.sync_copy(buf, acc.at[idx_d], add=True)
            return unused

        lax.fori_loop(0, _NCH, chunk_body, 0)
        plsc.subcore_barrier()

        pltpu.sync_copy(acc.at[pl.ds(r0, _RPT), :],
                        s_out.at[c, r, pl.ds(r0, _RPT), :])
        pltpu.sync_copy(deg.at[pl.ds(r0, _RPT)], d_out.at[c, r, pl.ds(r0, _RPT)])
        plsc.subcore_barrier()


def _segment_sums(table64, src5, dst5, zrows, zdeg, ones_h):
    f32 = jnp.float32
    run = pl.kernel(
        _sc_body,
        out_type=[
            jax.ShapeDtypeStruct((2, 2, _NPAD, _HID), f32),
            jax.ShapeDtypeStruct((2, 2, _NPAD), f32),
        ],
        mesh=plsc.VectorSubcoreMesh(core_axis_name="c", subcore_axis_name="s"),
        compiler_params=pltpu.CompilerParams(use_tc_tiling_on_sc=False,
                                             needs_layout_passes=False),
        scratch_types=[
            pltpu.VMEM_SHARED((_NPAD, _HID), f32),
            pltpu.VMEM_SHARED((_NPAD,), f32),
            pltpu.VMEM((_CH,), jnp.int32),
            pltpu.VMEM((_CH,), jnp.int32),
            pltpu.VMEM((_CH, _HID), f32),
            pltpu.VMEM((_CH,), f32),
            pltpu.SemaphoreType.DMA,
        ],
    )
    return run(table64, src5, dst5, zrows, zdeg, ones_h)


_BM_C = 2000


def _fin_body(s_ref, d_ref, benc_ref, attn_ref, wdec_ref, bdec_ref,
              z_ref, xh_ref):
    a0 = attn_ref[0]
    a1 = attn_ref[1]
    m = jnp.maximum(a0, a1)
    e0 = jnp.exp(a0 - m)
    e1 = jnp.exp(a1 - m)
    w0 = e0 / (e0 + e1)
    w1 = e1 / (e0 + e1)
    sr = s_ref[0]
    dr = d_ref[0]
    d0 = jnp.maximum(dr[0], 1.0)
    d1 = jnp.maximum(dr[1], 1.0)
    z0 = jnp.maximum(sr[0] / d0 + benc_ref[0:1, :], 0.0)
    z1 = jnp.maximum(sr[1] / d1 + benc_ref[1:2, :], 0.0)
    z = w0 * z0 + w1 * z1
    z_ref[...] = z
    xh_ref[...] = (jnp.dot(z, wdec_ref[...], preferred_element_type=jnp.float32)
                   + bdec_ref[0:1, :])


def _finalize(s_all, d_all, view, benc, attn, w_dec, bdec_2d):
    nb = _N // _BM_C
    return pl.pallas_call(
        _fin_body,
        grid=(nb,),
        in_specs=[
            pl.BlockSpec((1, 2, _BM_C, _HID), lambda i, v=view: (v, 0, i, 0)),
            pl.BlockSpec((1, 2, _BM_C, 1), lambda i, v=view: (v, 0, i, 0)),
            pl.BlockSpec((2, _HID), lambda i: (0, 0)),
            pl.BlockSpec(memory_space=pltpu.SMEM),
            pl.BlockSpec((_HID, _IN), lambda i: (0, 0)),
            pl.BlockSpec((1, _IN), lambda i: (0, 0)),
        ],
        out_specs=[
            pl.BlockSpec((_BM_C, _HID), lambda i: (i, 0)),
            pl.BlockSpec((_BM_C, _IN), lambda i: (i, 0)),
        ],
        out_shape=[
            jax.ShapeDtypeStruct((_N, _HID), jnp.float32),
            jax.ShapeDtypeStruct((_N, _IN), jnp.float32),
        ],
    )(s_all, d_all, benc, attn, w_dec, bdec_2d)


_BM_D = 512
_BN_D = 2048


def _adj_body(zi_ref, zj_ref, out_ref):
    x = lax.dot_general(zi_ref[...], zj_ref[...],
                        dimension_numbers=(((1,), (1,)), ((), ())),
                        preferred_element_type=jnp.float32)
    out_ref[...] = 1.0 / (1.0 + jnp.exp(-x))


def _adjacency(z):
    ni = pl.cdiv(_N, _BM_D)
    nj = pl.cdiv(_N, _BN_D)
    return pl.pallas_call(
        _adj_body,
        grid=(ni, nj),
        in_specs=[
            pl.BlockSpec((_BM_D, _HID), lambda i, j: (i, 0)),
            pl.BlockSpec((_BN_D, _HID), lambda i, j: (j, 0)),
        ],
        out_specs=pl.BlockSpec((_BM_D, _BN_D), lambda i, j: (i, j)),
        out_shape=jax.ShapeDtypeStruct((_N, _N), jnp.float32),
    )(z, z)


def kernel(x_view_A, edge_indices_A, x_view_B, edge_indices_B, W_proj, b_proj,
           W_enc0, b_enc0, W_enc1, b_enc1, attn_weights, W_dec, b_dec):
    f32 = jnp.float32

    x_stacked = jnp.stack([x_view_A, x_view_B])
    w_enc_stacked = jnp.stack([W_enc0, W_enc1])
    b_proj_2d = b_proj.reshape(1, _IN)
    bdec_2d = b_dec.reshape(1, _IN)
    benc = jnp.stack([b_enc0, b_enc1])

    ei = jnp.stack([edge_indices_A, edge_indices_B])
    voff = jnp.arange(2, dtype=jnp.int32).reshape(2, 1, 1) * (2 * _N)
    roff = jnp.arange(2, dtype=jnp.int32).reshape(1, 2, 1)
    src = 2 * ei[:, :, 0, :] + voff + roff
    dst = ei[:, :, 1, :]
    pad = _E2 - _E
    src = jnp.concatenate(
        [src, jnp.zeros((2, 2, pad), jnp.int32)], axis=2)
    dst = jnp.concatenate(
        [dst, jnp.full((2, 2, pad), _NPAD - 1, jnp.int32)], axis=2)
    nchunks = _NTILES * _NCH
    src5 = src.reshape(2, 2, nchunks, _CH)
    dst5 = dst.reshape(2, 2, nchunks, _CH)

    zrows = jnp.zeros((_RPT, _HID), f32)
    zdeg = jnp.zeros((_RPT,), f32)
    ones_h = jnp.ones((_CH,), f32)

    table = _build_table(x_stacked, W_proj, w_enc_stacked, b_proj_2d)
    table64 = table.reshape(4 * _N, _HID)

    s_all, d_all = _segment_sums(table64, src5, dst5, zrows, zdeg, ones_h)
    d_all4 = d_all.reshape(2, 2, _NPAD, 1)

    z_A, xh_A = _finalize(s_all, d_all4, 0, benc, attn_weights, W_dec, bdec_2d)
    z_B, xh_B = _finalize(s_all, d_all4, 1, benc, attn_weights, W_dec, bdec_2d)

    adj_A = _adjacency(z_A)
    adj_B = _adjacency(z_B)

    return ((xh_A, adj_A), (xh_B, adj_B), (z_A, z_B))


# speedup vs baseline: 6.2058x; 1.3044x over previous
import jax
import jax.numpy as jnp
from jax import lax
from jax.experimental import pallas as pl
from jax.experimental.pallas import tpu as pltpu
from jax.experimental.pallas import tpu_sc as plsc

_N = 10000
_E = 320000
_IN = 128
_HID = 64
_TW = 128

_NTILES = 16
_EPT = _E // _NTILES
_SL = 125
_K = 8
_KH = 2
_CH = _SL * _K
_NCH = _EPT // _CH
_NPAD = 10240
_RPT = _NPAD // _NTILES


_BM_A = 2000


def _proj_body(x_ref, wp_ref, we_ref, bp_ref, out_ref):
    xb = x_ref[0]
    we = we_ref[0]
    m = jnp.dot(wp_ref[...], we, preferred_element_type=jnp.float32)
    c = jnp.dot(bp_ref[...], we, preferred_element_type=jnp.float32)
    g = jnp.dot(xb, m, preferred_element_type=jnp.float32) + c
    col = lax.broadcasted_iota(jnp.int32, (_BM_A, _TW - _HID), 1)
    pat = (col == 0).astype(jnp.float32)
    out_ref[...] = jnp.concatenate([g, pat], axis=1)


def _build_table(x_stacked, w_proj, w_enc_stacked, b_proj_2d):
    nb = _N // _BM_A
    return pl.pallas_call(
        _proj_body,
        grid=(2, 2, nb),
        in_specs=[
            pl.BlockSpec((1, _BM_A, _IN), lambda v, r, i: (v, i, 0)),
            pl.BlockSpec((_IN, _IN), lambda v, r, i: (0, 0)),
            pl.BlockSpec((1, _IN, _HID), lambda v, r, i: (r, 0, 0)),
            pl.BlockSpec((1, _IN), lambda v, r, i: (0, 0)),
        ],
        out_specs=pl.BlockSpec(
            (_BM_A, _TW), lambda v, r, i: ((v * 2 + r) * nb + i, 0)),
        out_shape=jax.ShapeDtypeStruct((4 * _N, _TW), jnp.float32),
    )(x_stacked, w_proj, w_enc_stacked, b_proj_2d)


def _sc_body(table, src_h, dst_h, zrows,
             s_out,
             acc, idx_s, idx_d, buf0, buf1, sem):
    c = lax.axis_index("c")
    s = lax.axis_index("s")
    r0 = s * _RPT

    for r in range(2):
        pltpu.sync_copy(zrows, acc.at[pl.ds(r0, _RPT), :])
        plsc.subcore_barrier()

        def chunk_body(k, unused, r=r):
            q = s * _NCH + k
            bufs = (buf0, buf1)
            pltpu.sync_copy(src_h.at[c, r, q], idx_s)
            pltpu.sync_copy(dst_h.at[c, r, q], idx_d)
            cps = {0: pltpu.async_copy(table.at[idx_s.at[0]], bufs[0], sem)}
            for j in range(_K):
                cps[j].wait()
                if j + 1 < _K:
                    cps[j + 1] = pltpu.async_copy(
                        table.at[idx_s.at[j + 1]], bufs[(j + 1) % 2], sem)
                pltpu.sync_copy(bufs[j % 2], acc.at[idx_d.at[j]], add=True)
            return unused

        lax.fori_loop(0, _NCH, chunk_body, 0)
        plsc.subcore_barrier()

        pltpu.sync_copy(acc.at[pl.ds(r0, _RPT), :],
                        s_out.at[c, r, pl.ds(r0, _RPT), :])
        plsc.subcore_barrier()


def _segment_sums(table, src5, dst5, zrows):
    f32 = jnp.float32
    run = pl.kernel(
        _sc_body,
        out_type=jax.ShapeDtypeStruct((2, 2, _NPAD, _TW), f32),
        mesh=plsc.VectorSubcoreMesh(core_axis_name="c", subcore_axis_name="s"),
        scratch_types=[
            pltpu.VMEM_SHARED((_NPAD, _TW), f32),
            pltpu.VMEM((_K, _SL), jnp.int32),
            pltpu.VMEM((_K, _SL), jnp.int32),
            pltpu.VMEM((_SL, _TW), f32),
            pltpu.VMEM((_SL, _TW), f32),
            pltpu.SemaphoreType.DMA,
        ],
    )
    return run(table, src5, dst5, zrows)


_BM_C = 2000


def _fin_body(s_ref, benc_ref, attn_ref, wdec_ref, bdec_ref,
              z_ref, xh_ref):
    a0 = attn_ref[0]
    a1 = attn_ref[1]
    m = jnp.maximum(a0, a1)
    e0 = jnp.exp(a0 - m)
    e1 = jnp.exp(a1 - m)
    w0 = e0 / (e0 + e1)
    w1 = e1 / (e0 + e1)
    sr = s_ref[0]
    d0 = jnp.maximum(sr[0, :, _HID:_HID + 1], 1.0)
    d1 = jnp.maximum(sr[1, :, _HID:_HID + 1], 1.0)
    z0 = jnp.maximum(sr[0, :, 0:_HID] / d0 + benc_ref[0:1, :], 0.0)
    z1 = jnp.maximum(sr[1, :, 0:_HID] / d1 + benc_ref[1:2, :], 0.0)
    z = w0 * z0 + w1 * z1
    z_ref[...] = z
    xh_ref[...] = (jnp.dot(z, wdec_ref[...], preferred_element_type=jnp.float32)
                   + bdec_ref[0:1, :])


def _finalize(s_all, view, benc, attn, w_dec, bdec_2d):
    nb = _N // _BM_C
    return pl.pallas_call(
        _fin_body,
        grid=(nb,),
        in_specs=[
            pl.BlockSpec((1, 2, _BM_C, _TW), lambda i, v=view: (v, 0, i, 0)),
            pl.BlockSpec((2, _HID), lambda i: (0, 0)),
            pl.BlockSpec(memory_space=pltpu.SMEM),
            pl.BlockSpec((_HID, _IN), lambda i: (0, 0)),
            pl.BlockSpec((1, _IN), lambda i: (0, 0)),
        ],
        out_specs=[
            pl.BlockSpec((_BM_C, _HID), lambda i: (i, 0)),
            pl.BlockSpec((_BM_C, _IN), lambda i: (i, 0)),
        ],
        out_shape=[
            jax.ShapeDtypeStruct((_N, _HID), jnp.float32),
            jax.ShapeDtypeStruct((_N, _IN), jnp.float32),
        ],
    )(s_all, benc, attn, w_dec, bdec_2d)


_BM_D = 512
_BN_D = 2048


def _adj_body(zi_ref, zj_ref, out_ref):
    x = lax.dot_general(zi_ref[...], zj_ref[...],
                        dimension_numbers=(((1,), (1,)), ((), ())),
                        preferred_element_type=jnp.float32)
    out_ref[...] = 0.5 * jnp.tanh(0.5 * x) + 0.5


def _adjacency(z):
    ni = pl.cdiv(_N, _BM_D)
    nj = pl.cdiv(_N, _BN_D)
    return pl.pallas_call(
        _adj_body,
        grid=(ni, nj),
        in_specs=[
            pl.BlockSpec((_BM_D, _HID), lambda i, j: (i, 0)),
            pl.BlockSpec((_BN_D, _HID), lambda i, j: (j, 0)),
        ],
        out_specs=pl.BlockSpec((_BM_D, _BN_D), lambda i, j: (i, j)),
        out_shape=jax.ShapeDtypeStruct((_N, _N), jnp.float32),
    )(z, z)


def kernel(x_view_A, edge_indices_A, x_view_B, edge_indices_B, W_proj, b_proj,
           W_enc0, b_enc0, W_enc1, b_enc1, attn_weights, W_dec, b_dec):
    f32 = jnp.float32

    x_stacked = jnp.stack([x_view_A, x_view_B])
    w_enc_stacked = jnp.stack([W_enc0, W_enc1])
    b_proj_2d = b_proj.reshape(1, _IN)
    bdec_2d = b_dec.reshape(1, _IN)
    benc = jnp.stack([b_enc0, b_enc1])

    ei = jnp.stack([edge_indices_A, edge_indices_B])
    table_off = (jnp.arange(2, dtype=jnp.int32)[:, None, None] * 2
                 + jnp.arange(2, dtype=jnp.int32)[None, :, None]) * _N
    src = ei[:, :, 0, :] + table_off
    dst = ei[:, :, 1, :]
    nchunks = _NTILES * _NCH
    src5 = src.reshape(2, 2, nchunks, _K, _SL)
    dst5 = dst.reshape(2, 2, nchunks, _K, _SL)

    zrows = jnp.zeros((_RPT, _TW), f32)

    table = _build_table(x_stacked, W_proj, w_enc_stacked, b_proj_2d)

    s_all = _segment_sums(table, src5, dst5, zrows)

    z_A, xh_A = _finalize(s_all, 0, benc, attn_weights, W_dec, bdec_2d)
    z_B, xh_B = _finalize(s_all, 1, benc, attn_weights, W_dec, bdec_2d)

    adj_A = _adjacency(z_A)
    adj_B = _adjacency(z_B)

    return ((xh_A, adj_A), (xh_B, adj_B), (z_A, z_B))


# speedup vs baseline: 6.6299x; 1.0684x over previous
import jax
import jax.numpy as jnp
from jax import lax
from jax.experimental import pallas as pl
from jax.experimental.pallas import tpu as pltpu
from jax.experimental.pallas import tpu_sc as plsc

_N = 10000
_E = 320000
_IN = 128
_HID = 64
_TW = 128

_NTILES = 16
_EPT = _E // _NTILES
_SL = 125
_K = 8
_KH = 2
_CH = _SL * _K
_NCH = _EPT // _CH
_NPAD = 10240
_RPT = _NPAD // _NTILES


_BM_A = 2000


def _proj_body(x_ref, wp_ref, we_ref, bp_ref, out_ref):
    xb = x_ref[0]
    we = we_ref[0]
    m = jnp.dot(wp_ref[...], we, preferred_element_type=jnp.float32)
    c = jnp.dot(bp_ref[...], we, preferred_element_type=jnp.float32)
    g = jnp.dot(xb, m, preferred_element_type=jnp.float32) + c
    col = lax.broadcasted_iota(jnp.int32, (_BM_A, _TW - _HID), 1)
    pat = (col == 0).astype(jnp.float32)
    out_ref[...] = jnp.concatenate([g, pat], axis=1)


def _build_table(x_stacked, w_proj, w_enc_stacked, b_proj_2d):
    nb = _N // _BM_A
    return pl.pallas_call(
        _proj_body,
        grid=(2, 2, nb),
        in_specs=[
            pl.BlockSpec((1, _BM_A, _IN), lambda v, r, i: (v, i, 0)),
            pl.BlockSpec((_IN, _IN), lambda v, r, i: (0, 0)),
            pl.BlockSpec((1, _IN, _HID), lambda v, r, i: (r, 0, 0)),
            pl.BlockSpec((1, _IN), lambda v, r, i: (0, 0)),
        ],
        out_specs=pl.BlockSpec(
            (_BM_A, _TW), lambda v, r, i: ((v * 2 + r) * nb + i, 0)),
        out_shape=jax.ShapeDtypeStruct((4 * _N, _TW), jnp.float32),
    )(x_stacked, w_proj, w_enc_stacked, b_proj_2d)


def _sc_body(table, src_h, dst_h, zrows,
             s_out,
             acc, idx_s, idx_d, buf0, buf1, sem):
    c = lax.axis_index("c")
    s = lax.axis_index("s")
    r0 = s * _RPT

    for r in range(2):
        pltpu.sync_copy(zrows, acc.at[pl.ds(r0, _RPT), :])
        plsc.subcore_barrier()

        def chunk_body(k, unused, r=r):
            q = s * _NCH + k
            bufs = (buf0, buf1)
            pltpu.sync_copy(src_h.at[c, r, q], idx_s)
            pltpu.sync_copy(dst_h.at[c, r, q], idx_d)
            cps = {0: pltpu.async_copy(table.at[idx_s.at[0]], bufs[0], sem)}
            for j in range(_K):
                cps[j].wait()
                if j + 1 < _K:
                    cps[j + 1] = pltpu.async_copy(
                        table.at[idx_s.at[j + 1]], bufs[(j + 1) % 2], sem)
                pltpu.sync_copy(bufs[j % 2], acc.at[idx_d.at[j]], add=True)
            return unused

        lax.fori_loop(0, _NCH, chunk_body, 0)
        plsc.subcore_barrier()

        pltpu.sync_copy(acc.at[pl.ds(r0, _RPT), :],
                        s_out.at[c, r, pl.ds(r0, _RPT), :])
        plsc.subcore_barrier()


def _segment_sums(table, src5, dst5, zrows):
    f32 = jnp.float32
    run = pl.kernel(
        _sc_body,
        out_type=jax.ShapeDtypeStruct((2, 2, _NPAD, _TW), f32),
        mesh=plsc.VectorSubcoreMesh(core_axis_name="c", subcore_axis_name="s"),
        scratch_types=[
            pltpu.VMEM_SHARED((_NPAD, _TW), f32),
            pltpu.VMEM((_K, _SL), jnp.int32),
            pltpu.VMEM((_K, _SL), jnp.int32),
            pltpu.VMEM((_SL, _TW), f32),
            pltpu.VMEM((_SL, _TW), f32),
            pltpu.SemaphoreType.DMA,
        ],
    )
    return run(table, src5, dst5, zrows)


_BM_C = 2000


def _fin_body(s_ref, benc_ref, attn_ref, wdec_ref, bdec_ref,
              z_ref, xh_ref):
    a0 = attn_ref[0]
    a1 = attn_ref[1]
    m = jnp.maximum(a0, a1)
    e0 = jnp.exp(a0 - m)
    e1 = jnp.exp(a1 - m)
    w0 = e0 / (e0 + e1)
    w1 = e1 / (e0 + e1)
    sr = s_ref[0]
    d0 = jnp.maximum(sr[0, :, _HID:_HID + 1], 1.0)
    d1 = jnp.maximum(sr[1, :, _HID:_HID + 1], 1.0)
    z0 = jnp.maximum(sr[0, :, 0:_HID] / d0 + benc_ref[0:1, :], 0.0)
    z1 = jnp.maximum(sr[1, :, 0:_HID] / d1 + benc_ref[1:2, :], 0.0)
    z = w0 * z0 + w1 * z1
    z_ref[...] = z
    xh_ref[...] = (jnp.dot(z, wdec_ref[...], preferred_element_type=jnp.float32)
                   + bdec_ref[0:1, :])


def _finalize(s_all, view, benc, attn, w_dec, bdec_2d):
    nb = _N // _BM_C
    return pl.pallas_call(
        _fin_body,
        grid=(nb,),
        in_specs=[
            pl.BlockSpec((1, 2, _BM_C, _TW), lambda i, v=view: (v, 0, i, 0)),
            pl.BlockSpec((2, _HID), lambda i: (0, 0)),
            pl.BlockSpec(memory_space=pltpu.SMEM),
            pl.BlockSpec((_HID, _IN), lambda i: (0, 0)),
            pl.BlockSpec((1, _IN), lambda i: (0, 0)),
        ],
        out_specs=[
            pl.BlockSpec((_BM_C, _HID), lambda i: (i, 0)),
            pl.BlockSpec((_BM_C, _IN), lambda i: (i, 0)),
        ],
        out_shape=[
            jax.ShapeDtypeStruct((_N, _HID), jnp.float32),
            jax.ShapeDtypeStruct((_N, _IN), jnp.float32),
        ],
    )(s_all, benc, attn, w_dec, bdec_2d)


_BM_D = 1024
_BN_D = 2048


def _adj_body(zi_ref, zj_ref, out_ref):
    x = lax.dot_general(zi_ref[...], zj_ref[...],
                        dimension_numbers=(((1,), (1,)), ((), ())),
                        preferred_element_type=jnp.float32)
    out_ref[...] = 0.5 * jnp.tanh(0.5 * x) + 0.5


def _adjacency(z):
    ni = pl.cdiv(_N, _BM_D)
    nj = pl.cdiv(_N, _BN_D)
    return pl.pallas_call(
        _adj_body,
        grid=(ni, nj),
        in_specs=[
            pl.BlockSpec((_BM_D, _HID), lambda i, j: (i, 0)),
            pl.BlockSpec((_BN_D, _HID), lambda i, j: (j, 0)),
        ],
        out_specs=pl.BlockSpec((_BM_D, _BN_D), lambda i, j: (i, j)),
        out_shape=jax.ShapeDtypeStruct((_N, _N), jnp.float32),
    )(z, z)


def kernel(x_view_A, edge_indices_A, x_view_B, edge_indices_B, W_proj, b_proj,
           W_enc0, b_enc0, W_enc1, b_enc1, attn_weights, W_dec, b_dec):
    f32 = jnp.float32

    x_stacked = jnp.stack([x_view_A, x_view_B])
    w_enc_stacked = jnp.stack([W_enc0, W_enc1])
    b_proj_2d = b_proj.reshape(1, _IN)
    bdec_2d = b_dec.reshape(1, _IN)
    benc = jnp.stack([b_enc0, b_enc1])

    ei = jnp.stack([edge_indices_A, edge_indices_B])
    table_off = (jnp.arange(2, dtype=jnp.int32)[:, None, None] * 2
                 + jnp.arange(2, dtype=jnp.int32)[None, :, None]) * _N
    src = ei[:, :, 0, :] + table_off
    dst = ei[:, :, 1, :]
    nchunks = _NTILES * _NCH
    src5 = src.reshape(2, 2, nchunks, _K, _SL)
    dst5 = dst.reshape(2, 2, nchunks, _K, _SL)

    zrows = jnp.zeros((_RPT, _TW), f32)

    table = _build_table(x_stacked, W_proj, w_enc_stacked, b_proj_2d)

    s_all = _segment_sums(table, src5, dst5, zrows)

    z_A, xh_A = _finalize(s_all, 0, benc, attn_weights, W_dec, bdec_2d)
    z_B, xh_B = _finalize(s_all, 1, benc, attn_weights, W_dec, bdec_2d)

    adj_A = _adjacency(z_A)
    adj_B = _adjacency(z_B)

    return ((xh_A, adj_A), (xh_B, adj_B), (z_A, z_B))


# speedup vs baseline: 6.8208x; 1.0288x over previous
import jax
import jax.numpy as jnp
from jax import lax
from jax.experimental import pallas as pl
from jax.experimental.pallas import tpu as pltpu
from jax.experimental.pallas import tpu_sc as plsc

_N = 10000
_E = 320000
_IN = 128
_HID = 64
_TW = 128

_NTILES = 16
_EPT = _E // _NTILES
_SL = 125
_K = 8
_KH = 2
_CH = _SL * _K
_NCH = _EPT // _CH
_NPAD = 10240
_RPT = _NPAD // _NTILES


_BM_A = 2000


def _proj_body(x_ref, wp_ref, we_ref, bp_ref, out_ref):
    xb = x_ref[0]
    we = we_ref[0]
    m = jnp.dot(wp_ref[...], we, preferred_element_type=jnp.float32)
    c = jnp.dot(bp_ref[...], we, preferred_element_type=jnp.float32)
    g = jnp.dot(xb, m, preferred_element_type=jnp.float32) + c
    col = lax.broadcasted_iota(jnp.int32, (_BM_A, _TW - _HID), 1)
    pat = (col == 0).astype(jnp.float32)
    out_ref[...] = jnp.concatenate([g, pat], axis=1)


def _build_table(x_stacked, w_proj, w_enc_stacked, b_proj_2d):
    nb = _N // _BM_A
    return pl.pallas_call(
        _proj_body,
        grid=(2, 2, nb),
        in_specs=[
            pl.BlockSpec((1, _BM_A, _IN), lambda v, r, i: (v, i, 0)),
            pl.BlockSpec((_IN, _IN), lambda v, r, i: (0, 0)),
            pl.BlockSpec((1, _IN, _HID), lambda v, r, i: (r, 0, 0)),
            pl.BlockSpec((1, _IN), lambda v, r, i: (0, 0)),
        ],
        out_specs=pl.BlockSpec(
            (_BM_A, _TW), lambda v, r, i: ((v * 2 + r) * nb + i, 0)),
        out_shape=jax.ShapeDtypeStruct((4 * _N, _TW), jnp.float32),
    )(x_stacked, w_proj, w_enc_stacked, b_proj_2d)


def _sc_body(table, src_h, dst_h, zrows,
             s_out,
             acc, idx_s, idx_d, buf0, buf1, sem):
    c = lax.axis_index("c")
    s = lax.axis_index("s")
    r0 = s * _RPT

    for r in range(2):
        pltpu.sync_copy(zrows, acc.at[pl.ds(r0, _RPT), :])
        plsc.subcore_barrier()

        def chunk_body(k, unused, r=r):
            q = s * _NCH + k
            bufs = (buf0, buf1)
            pltpu.sync_copy(src_h.at[c, r, q], idx_s)
            pltpu.sync_copy(dst_h.at[c, r, q], idx_d)
            cps = {0: pltpu.async_copy(table.at[idx_s.at[0]], bufs[0], sem)}
            for j in range(_K):
                cps[j].wait()
                if j + 1 < _K:
                    cps[j + 1] = pltpu.async_copy(
                        table.at[idx_s.at[j + 1]], bufs[(j + 1) % 2], sem)
                pltpu.sync_copy(bufs[j % 2], acc.at[idx_d.at[j]], add=True)
            return unused

        lax.fori_loop(0, _NCH, chunk_body, 0)
        plsc.subcore_barrier()

        pltpu.sync_copy(acc.at[pl.ds(r0, _RPT), :],
                        s_out.at[c, r, pl.ds(r0, _RPT), :])
        plsc.subcore_barrier()


def _segment_sums(table, src5, dst5, zrows):
    f32 = jnp.float32
    run = pl.kernel(
        _sc_body,
        out_type=jax.ShapeDtypeStruct((2, 2, _NPAD, _TW), f32),
        mesh=plsc.VectorSubcoreMesh(core_axis_name="c", subcore_axis_name="s"),
        scratch_types=[
            pltpu.VMEM_SHARED((_NPAD, _TW), f32),
            pltpu.VMEM((_K, _SL), jnp.int32),
            pltpu.VMEM((_K, _SL), jnp.int32),
            pltpu.VMEM((_SL, _TW), f32),
            pltpu.VMEM((_SL, _TW), f32),
            pltpu.SemaphoreType.DMA,
        ],
    )
    return run(table, src5, dst5, zrows)


_BM_C = 2000


def _fin_body(s_ref, benc_ref, attn_ref, wdec_ref, bdec_ref,
              z_ref, xh_ref):
    a0 = attn_ref[0]
    a1 = attn_ref[1]
    m = jnp.maximum(a0, a1)
    e0 = jnp.exp(a0 - m)
    e1 = jnp.exp(a1 - m)
    w0 = e0 / (e0 + e1)
    w1 = e1 / (e0 + e1)
    sr = s_ref[0]
    d0 = jnp.maximum(sr[0, :, _HID:_HID + 1], 1.0)
    d1 = jnp.maximum(sr[1, :, _HID:_HID + 1], 1.0)
    z0 = jnp.maximum(sr[0, :, 0:_HID] / d0 + benc_ref[0:1, :], 0.0)
    z1 = jnp.maximum(sr[1, :, 0:_HID] / d1 + benc_ref[1:2, :], 0.0)
    z = w0 * z0 + w1 * z1
    z_ref[...] = z
    xh_ref[...] = (jnp.dot(z, wdec_ref[...], preferred_element_type=jnp.float32)
                   + bdec_ref[0:1, :])


def _finalize(s_all, view, benc, attn, w_dec, bdec_2d):
    nb = _N // _BM_C
    return pl.pallas_call(
        _fin_body,
        grid=(nb,),
        in_specs=[
            pl.BlockSpec((1, 2, _BM_C, _TW), lambda i, v=view: (v, 0, i, 0)),
            pl.BlockSpec((2, _HID), lambda i: (0, 0)),
            pl.BlockSpec(memory_space=pltpu.SMEM),
            pl.BlockSpec((_HID, _IN), lambda i: (0, 0)),
            pl.BlockSpec((1, _IN), lambda i: (0, 0)),
        ],
        out_specs=[
            pl.BlockSpec((_BM_C, _HID), lambda i: (i, 0)),
            pl.BlockSpec((_BM_C, _IN), lambda i: (i, 0)),
        ],
        out_shape=[
            jax.ShapeDtypeStruct((_N, _HID), jnp.float32),
            jax.ShapeDtypeStruct((_N, _IN), jnp.float32),
        ],
    )(s_all, benc, attn, w_dec, bdec_2d)


_BM_D = 2048
_BN_D = 2048


def _adj_body(zi_ref, zj_ref, out_ref):
    x = lax.dot_general(zi_ref[...], zj_ref[...],
                        dimension_numbers=(((1,), (1,)), ((), ())),
                        preferred_element_type=jnp.float32)
    out_ref[...] = 0.5 * jnp.tanh(0.5 * x) + 0.5


def _adjacency(z):
    ni = pl.cdiv(_N, _BM_D)
    nj = pl.cdiv(_N, _BN_D)
    return pl.pallas_call(
        _adj_body,
        grid=(ni, nj),
        in_specs=[
            pl.BlockSpec((_BM_D, _HID), lambda i, j: (i, 0)),
            pl.BlockSpec((_BN_D, _HID), lambda i, j: (j, 0)),
        ],
        out_specs=pl.BlockSpec((_BM_D, _BN_D), lambda i, j: (i, j)),
        out_shape=jax.ShapeDtypeStruct((_N, _N), jnp.float32),
    )(z, z)


def kernel(x_view_A, edge_indices_A, x_view_B, edge_indices_B, W_proj, b_proj,
           W_enc0, b_enc0, W_enc1, b_enc1, attn_weights, W_dec, b_dec):
    f32 = jnp.float32

    x_stacked = jnp.stack([x_view_A, x_view_B])
    w_enc_stacked = jnp.stack([W_enc0, W_enc1])
    b_proj_2d = b_proj.reshape(1, _IN)
    bdec_2d = b_dec.reshape(1, _IN)
    benc = jnp.stack([b_enc0, b_enc1])

    ei = jnp.stack([edge_indices_A, edge_indices_B])
    table_off = (jnp.arange(2, dtype=jnp.int32)[:, None, None] * 2
                 + jnp.arange(2, dtype=jnp.int32)[None, :, None]) * _N
    src = ei[:, :, 0, :] + table_off
    dst = ei[:, :, 1, :]
    nchunks = _NTILES * _NCH
    src5 = src.reshape(2, 2, nchunks, _K, _SL)
    dst5 = dst.reshape(2, 2, nchunks, _K, _SL)

    zrows = jnp.zeros((_RPT, _TW), f32)

    table = _build_table(x_stacked, W_proj, w_enc_stacked, b_proj_2d)

    s_all = _segment_sums(table, src5, dst5, zrows)

    z_A, xh_A = _finalize(s_all, 0, benc, attn_weights, W_dec, bdec_2d)
    z_B, xh_B = _finalize(s_all, 1, benc, attn_weights, W_dec, bdec_2d)

    adj_A = _adjacency(z_A)
    adj_B = _adjacency(z_B)

    return ((xh_A, adj_A), (xh_B, adj_B), (z_A, z_B))
